# baseline (speedup 1.0000x reference)
import functools

import jax
import jax.numpy as jnp
from jax import lax
from jax.experimental import pallas as pl
from jax.experimental.pallas import tpu as pltpu
from jax.experimental.pallas import tpu_sc as plsc

N = 10000
HID = 128
NTILE = 16
CH = 128
NP = 10240
RPT = NP // NTILE


def _bcast_lane(v16, k):
    idx = jnp.full((16, 1), k, dtype=jnp.int32)
    return lax.gather(
        v16, idx,
        lax.GatherDimensionNumbers(
            offset_dims=(), collapsed_slice_dims=(0,), start_index_map=(0,)),
        (1,), mode=lax.GatherScatterMode.PROMISE_IN_BOUNDS)


def _norm_body(cpt, gidx_hbm, ew_hbm, nrm_hbm, dacc, zb, gi_v, ew_v, dg_v, o_v):
    c = lax.axis_index("c")
    s = lax.axis_index("s")
    zeros16 = jnp.zeros((16,), jnp.float32)

    def zrow(i, _):
        zb[pl.ds(i * 16, 16)] = zeros16
        return 0
    lax.fori_loop(0, RPT // 16, zrow, 0)
    pltpu.sync_copy(zb, dacc.at[pl.ds(s * RPT, RPT)])
    plsc.subcore_barrier()

    def chunk(g, _):
        off = (s * cpt + g) * CH
        pltpu.sync_copy(gidx_hbm.at[c, pl.ds(off, CH)], gi_v)
        pltpu.sync_copy(ew_hbm.at[pl.ds(off, CH)], ew_v)
        pltpu.sync_copy(ew_v, dacc.at[gi_v], add=True)
        return 0
    lax.fori_loop(0, cpt, chunk, 0)
    plsc.subcore_barrier()

    def chunk2(g, _):
        off = (s * cpt + g) * CH
        pltpu.sync_copy(gidx_hbm.at[c, pl.ds(off, CH)], gi_v)
        pltpu.sync_copy(ew_hbm.at[pl.ds(off, CH)], ew_v)
        pltpu.sync_copy(dacc.at[gi_v], dg_v)

        def grp(q, _):
            d16 = dg_v[pl.ds(q * 16, 16)]
            e16 = ew_v[pl.ds(q * 16, 16)]
            o_v[pl.ds(q * 16, 16)] = jnp.where(d16 > 0.0, e16 / d16, 0.0)
            return 0
        lax.fori_loop(0, 8, grp, 0)
        pltpu.sync_copy(o_v, nrm_hbm.at[c, pl.ds(off, CH)])
        return 0
    lax.fori_loop(0, cpt, chunk2, 0)


def _make_norm_kernel(ep):
    cpt = ep // (NTILE * CH)
    mesh = plsc.VectorSubcoreMesh(core_axis_name="c", subcore_axis_name="s")
    return functools.partial(
        pl.kernel,
        functools.partial(_norm_body, cpt),
        out_type=jax.ShapeDtypeStruct((2, ep), jnp.float32),
        mesh=mesh,
        scratch_types=[
            pltpu.VMEM_SHARED((NP,), jnp.float32),
            pltpu.VMEM((RPT,), jnp.float32),
            pltpu.VMEM((CH,), jnp.int32),
            pltpu.VMEM((CH,), jnp.float32),
            pltpu.VMEM((CH,), jnp.float32),
            pltpu.VMEM((CH,), jnp.float32),
        ],
    )()


def _prop_body(cpt, v_hbm, gidx_hbm, sidx_hbm, nrm_hbm, out_hbm,
               acc, rowbuf, gi_v, si_v, nr_v, sem):
    c = lax.axis_index("c")
    s = lax.axis_index("s")
    zeros16 = jnp.zeros((16,), jnp.float32)

    def zrow(i, _):
        for j in range(8):
            rowbuf[i, pl.ds(j * 16, 16)] = zeros16
        return 0
    lax.fori_loop(0, CH, zrow, 0)
    for r in range(RPT // CH):
        pltpu.sync_copy(rowbuf, acc.at[pl.ds(s * RPT + r * CH, CH)])
    plsc.subcore_barrier()

    def chunk(g, _):
        off = (s * cpt + g) * CH
        pltpu.sync_copy(gidx_hbm.at[c, pl.ds(off, CH)], gi_v)
        pltpu.sync_copy(sidx_hbm.at[c, pl.ds(off, CH)], si_v)
        pltpu.sync_copy(nrm_hbm.at[c, pl.ds(off, CH)], nr_v)
        pltpu.async_copy(v_hbm.at[gi_v], rowbuf, sem).wait()

        def grp(q, _):
            n16 = nr_v[pl.ds(q * 16, 16)]
            for k in range(16):
                b = _bcast_lane(n16, k)
                e = q * 16 + k
                for j in range(8):
                    sl = pl.ds(j * 16, 16)
                    rowbuf[e, sl] = rowbuf[e, sl] * b
            return 0
        lax.fori_loop(0, 8, grp, 0)
        pltpu.sync_copy(rowbuf, acc.at[si_v], add=True)
        return 0
    lax.fori_loop(0, cpt, chunk, 0)
    plsc.subcore_barrier()

    pltpu.sync_copy(acc.at[pl.ds(s * RPT, RPT)],
                    out_hbm.at[c, pl.ds(s * RPT, RPT)])


def _make_prop_kernel(ep):
    cpt = ep // (NTILE * CH)
    mesh = plsc.VectorSubcoreMesh(core_axis_name="c", subcore_axis_name="s")
    return functools.partial(
        pl.kernel,
        functools.partial(_prop_body, cpt),
        out_type=jax.ShapeDtypeStruct((2, NP, HID), jnp.float32),
        mesh=mesh,
        scratch_types=[
            pltpu.VMEM_SHARED((NP, HID), jnp.float32),
            pltpu.VMEM((CH, HID), jnp.float32),
            pltpu.VMEM((CH,), jnp.int32),
            pltpu.VMEM((CH,), jnp.int32),
            pltpu.VMEM((CH,), jnp.float32),
            pltpu.SemaphoreType.DMA,
        ],
    )()


def _gates_body(x, h, pox, poh, pix, pih, w, b, z_ref, q_ref):
    def mm(v, lo):
        return jax.lax.dot_general(
            v[...], w[pl.ds(lo, HID), :], (((1,), (0,)), ((), ())),
            preferred_element_type=jnp.float32)
    acc = (mm(x, 0) + mm(h, HID) + mm(pox, 2 * HID) + mm(poh, 3 * HID)
           + mm(pix, 4 * HID) + mm(pih, 5 * HID)) + b[...]
    zr = jax.nn.sigmoid(acc)
    z_ref[...] = zr[:, :HID]
    q_ref[...] = zr[:, HID:] * h[...]


def _cand_body(x, q, pox, poq, pix, piq, z, h, w, b, o_ref):
    def mm(v, lo):
        return jax.lax.dot_general(
            v[...], w[pl.ds(lo, HID), :], (((1,), (0,)), ((), ())),
            preferred_element_type=jnp.float32)
    acc = (mm(x, 0) + mm(q, HID) + mm(pox, 2 * HID) + mm(poq, 3 * HID)
           + mm(pix, 4 * HID) + mm(piq, 5 * HID)) + b[...]
    ht = jnp.tanh(acc)
    zv = z[...]
    o_ref[...] = jnp.maximum(zv * h[...] + (1.0 - zv) * ht, 0.0)


_BLK = 2000
_GRID = N // _BLK


def _row_spec(width):
    return pl.BlockSpec((_BLK, width), lambda i: (i, 0))


def _full_spec(shape):
    return pl.BlockSpec(shape, lambda i: (0,) * len(shape))


def _gates_call(x, h, pox, poh, pix, pih, w, b):
    return pl.pallas_call(
        _gates_body,
        grid=(_GRID,),
        in_specs=[_row_spec(HID)] * 6 + [_full_spec(w.shape), _full_spec(b.shape)],
        out_specs=[_row_spec(HID), _row_spec(HID)],
        out_shape=[jax.ShapeDtypeStruct((N, HID), jnp.float32)] * 2,
    )(x, h, pox, poh, pix, pih, w, b)


def _cand_call(x, q, pox, poq, pix, piq, z, h, w, b):
    return pl.pallas_call(
        _cand_body,
        grid=(_GRID,),
        in_specs=[_row_spec(HID)] * 8 + [_full_spec(w.shape), _full_spec(b.shape)],
        out_specs=_row_spec(HID),
        out_shape=jax.ShapeDtypeStruct((N, HID), jnp.float32),
    )(x, q, pox, poq, pix, piq, z, h, w, b)


def _final_linear_body(h_ref, w_ref, b_ref, o_ref):
    o_ref[...] = h_ref[...] @ w_ref[...] + b_ref[...][None, :]


def _pack_gate_weight(w):
    s = w[0, 0] + w[1, 0]
    return jnp.concatenate(
        [s[:HID], s[HID:], w[0, 1][:HID], w[0, 1][HID:],
         w[1, 1][:HID], w[1, 1][HID:]], axis=0)


def kernel(x, edge_index, edge_weight, w1_z, b1_z, w1_r, b1_r, w1_h, b1_h,
           w2_z, b2_z, w2_r, b2_r, w2_h, b2_h, lin_w, lin_b):
    B, n, f, T_ = x.shape
    E = edge_weight.shape[0]
    quant = NTILE * CH * 8
    ep = ((E + quant - 1) // quant) * quant
    pad = ep - E

    src = edge_index[0].astype(jnp.int32)
    dst = edge_index[1].astype(jnp.int32)
    zpi = jnp.zeros((pad,), jnp.int32)
    src_p = jnp.concatenate([src, zpi])
    dst_p = jnp.concatenate([dst, zpi])
    ew_p = jnp.concatenate([edge_weight, jnp.zeros((pad,), jnp.float32)])
    gidx = jnp.stack([src_p, dst_p])
    sidx = jnp.stack([dst_p, src_p])

    nrm = _make_norm_kernel(ep)(gidx, ew_p)

    prop_k = _make_prop_kernel(ep)

    def prop(v):
        out = prop_k(v, gidx, sidx, nrm)
        return out[0, :N], out[1, :N]

    xs = jnp.moveaxis(x[0], -1, 0)

    wz1 = jnp.concatenate([_pack_gate_weight(w1_z), _pack_gate_weight(w1_r)], axis=1)
    bz1 = jnp.concatenate([b1_z, b1_r]).reshape(1, 2 * HID)
    wh1 = _pack_gate_weight(w1_h)
    bh1 = b1_h.reshape(1, HID)
    wz2 = jnp.concatenate([_pack_gate_weight(w2_z), _pack_gate_weight(w2_r)], axis=1)
    bz2 = jnp.concatenate([b2_z, b2_r]).reshape(1, 2 * HID)
    wh2 = _pack_gate_weight(w2_h)
    bh2 = b2_h.reshape(1, HID)

    h1 = jnp.zeros((N, HID), jnp.float32)
    h2 = jnp.zeros((N, HID), jnp.float32)
    for t in range(T_):
        xt = xs[t]
        pox, pix = prop(xt)
        poh, pih = prop(h1)
        z1, q1 = _gates_call(xt, h1, pox, poh, pix, pih, wz1, bz1)
        poq, piq = prop(q1)
        h1 = _cand_call(xt, q1, pox, poq, pix, piq, z1, h1, wh1, bh1)

        pox2, pix2 = prop(h1)
        poh2, pih2 = prop(h2)
        z2, q2 = _gates_call(h1, h2, pox2, poh2, pix2, pih2, wz2, bz2)
        poq2, piq2 = prop(q2)
        h2 = _cand_call(h1, q2, pox2, poq2, pix2, piq2, z2, h2, wh2, bh2)

    out = pl.pallas_call(
        _final_linear_body,
        out_shape=jax.ShapeDtypeStruct((N, lin_w.shape[1]), jnp.float32),
    )(h2, lin_w, lin_b)
    return out.reshape(B, n, -1)


# speedup vs baseline: 1.5797x; 1.5797x over previous
import functools

import jax
import jax.numpy as jnp
from jax import lax
from jax.experimental import pallas as pl
from jax.experimental.pallas import tpu as pltpu
from jax.experimental.pallas import tpu_sc as plsc

N = 10000
HID = 128
NTILE = 16
CH = 128
NP = 10240
RPT = NP // NTILE


def _bcast_lane(v16, k):
    idx = jnp.full((16, 1), k, dtype=jnp.int32)
    return lax.gather(
        v16, idx,
        lax.GatherDimensionNumbers(
            offset_dims=(), collapsed_slice_dims=(0,), start_index_map=(0,)),
        (1,), mode=lax.GatherScatterMode.PROMISE_IN_BOUNDS)


def _norm_body(cpt, gidx_hbm, ew_hbm, nrm_hbm, dacc, zb, gi_v, ew_v, dg_v, o_v):
    c = lax.axis_index("c")
    s = lax.axis_index("s")
    zeros16 = jnp.zeros((16,), jnp.float32)

    def zrow(i, _):
        zb[pl.ds(i * 16, 16)] = zeros16
        return 0
    lax.fori_loop(0, RPT // 16, zrow, 0)
    pltpu.sync_copy(zb, dacc.at[pl.ds(s * RPT, RPT)])
    plsc.subcore_barrier()

    def chunk(g, _):
        off = (s * cpt + g) * CH
        pltpu.sync_copy(gidx_hbm.at[c, pl.ds(off, CH)], gi_v)
        pltpu.sync_copy(ew_hbm.at[pl.ds(off, CH)], ew_v)
        pltpu.sync_copy(ew_v, dacc.at[gi_v], add=True)
        return 0
    lax.fori_loop(0, cpt, chunk, 0)
    plsc.subcore_barrier()

    def chunk2(g, _):
        off = (s * cpt + g) * CH
        pltpu.sync_copy(gidx_hbm.at[c, pl.ds(off, CH)], gi_v)
        pltpu.sync_copy(ew_hbm.at[pl.ds(off, CH)], ew_v)
        pltpu.sync_copy(dacc.at[gi_v], dg_v)

        def grp(q, _):
            d16 = dg_v[pl.ds(q * 16, 16)]
            e16 = ew_v[pl.ds(q * 16, 16)]
            o_v[pl.ds(q * 16, 16)] = jnp.where(d16 > 0.0, e16 / d16, 0.0)
            return 0
        lax.fori_loop(0, 8, grp, 0)
        pltpu.sync_copy(o_v, nrm_hbm.at[c, pl.ds(off, CH)])
        return 0
    lax.fori_loop(0, cpt, chunk2, 0)


def _make_norm_kernel(ep):
    cpt = ep // (NTILE * CH)
    mesh = plsc.VectorSubcoreMesh(core_axis_name="c", subcore_axis_name="s")
    return functools.partial(
        pl.kernel,
        functools.partial(_norm_body, cpt),
        out_type=jax.ShapeDtypeStruct((2, ep), jnp.float32),
        mesh=mesh,
        scratch_types=[
            pltpu.VMEM_SHARED((NP,), jnp.float32),
            pltpu.VMEM((RPT,), jnp.float32),
            pltpu.VMEM((CH,), jnp.int32),
            pltpu.VMEM((CH,), jnp.float32),
            pltpu.VMEM((CH,), jnp.float32),
            pltpu.VMEM((CH,), jnp.float32),
        ],
    )()


def _prop_body(cpt, v_hbm, gidx_hbm, sidx_hbm, nrm_hbm, out_hbm,
               acc, rowbuf, gi_v, si_v, nr_v, sem):
    c = lax.axis_index("c")
    s = lax.axis_index("s")
    zeros16 = jnp.zeros((16,), jnp.float32)

    def zrow(i, _):
        for j in range(8):
            rowbuf[i, pl.ds(j * 16, 16)] = zeros16
        return 0
    lax.fori_loop(0, CH, zrow, 0)
    for r in range(RPT // CH):
        pltpu.sync_copy(rowbuf, acc.at[pl.ds(s * RPT + r * CH, CH)])
    plsc.subcore_barrier()

    def chunk(g, _):
        off = (s * cpt + g) * CH
        pltpu.sync_copy(gidx_hbm.at[c, pl.ds(off, CH)], gi_v)
        pltpu.sync_copy(sidx_hbm.at[c, pl.ds(off, CH)], si_v)
        pltpu.sync_copy(nrm_hbm.at[c, pl.ds(off, CH)], nr_v)
        pltpu.async_copy(v_hbm.at[gi_v], rowbuf, sem).wait()

        def grp(q, _):
            n16 = nr_v[pl.ds(q * 16, 16)]
            for k in range(16):
                b = _bcast_lane(n16, k)
                e = q * 16 + k
                for j in range(8):
                    sl = pl.ds(j * 16, 16)
                    rowbuf[e, sl] = rowbuf[e, sl] * b
            return 0
        lax.fori_loop(0, 8, grp, 0)
        pltpu.sync_copy(rowbuf, acc.at[si_v], add=True)
        return 0
    lax.fori_loop(0, cpt, chunk, 0)
    plsc.subcore_barrier()

    pltpu.sync_copy(acc.at[pl.ds(s * RPT, RPT)],
                    out_hbm.at[c, pl.ds(s * RPT, RPT)])


def _make_prop_kernel(ep):
    cpt = ep // (NTILE * CH)
    mesh = plsc.VectorSubcoreMesh(core_axis_name="c", subcore_axis_name="s")
    return functools.partial(
        pl.kernel,
        functools.partial(_prop_body, cpt),
        out_type=jax.ShapeDtypeStruct((2, NP, HID), jnp.float32),
        mesh=mesh,
        scratch_types=[
            pltpu.VMEM_SHARED((NP, HID), jnp.float32),
            pltpu.VMEM((CH, HID), jnp.float32),
            pltpu.VMEM((CH,), jnp.int32),
            pltpu.VMEM((CH,), jnp.int32),
            pltpu.VMEM((CH,), jnp.float32),
            pltpu.SemaphoreType.DMA,
        ],
    )()


def _gates_body(x, h, pox, poh, pix, pih, w, b, z_ref, q_ref):
    def mm(v, lo):
        return jax.lax.dot_general(
            v[...], w[pl.ds(lo, HID), :], (((1,), (0,)), ((), ())),
            preferred_element_type=jnp.float32)
    acc = (mm(x, 0) + mm(h, HID) + mm(pox, 2 * HID) + mm(poh, 3 * HID)
           + mm(pix, 4 * HID) + mm(pih, 5 * HID)) + b[...]
    zr = jax.nn.sigmoid(acc)
    z_ref[...] = zr[:, :HID]
    q_ref[...] = zr[:, HID:] * h[...]


def _cand_body(x, q, pox, poq, pix, piq, z, h, w, b, o_ref):
    def mm(v, lo):
        return jax.lax.dot_general(
            v[...], w[pl.ds(lo, HID), :], (((1,), (0,)), ((), ())),
            preferred_element_type=jnp.float32)
    acc = (mm(x, 0) + mm(q, HID) + mm(pox, 2 * HID) + mm(poq, 3 * HID)
           + mm(pix, 4 * HID) + mm(piq, 5 * HID)) + b[...]
    ht = jnp.tanh(acc)
    zv = z[...]
    o_ref[...] = jnp.maximum(zv * h[...] + (1.0 - zv) * ht, 0.0)


_BLK = 2000
_GRID = N // _BLK


def _row_spec(width):
    return pl.BlockSpec((_BLK, width), lambda i: (i, 0))


def _full_spec(shape):
    return pl.BlockSpec(shape, lambda i: (0,) * len(shape))


def _gates_call(x, h, pox, poh, pix, pih, w, b):
    return pl.pallas_call(
        _gates_body,
        grid=(_GRID,),
        in_specs=[_row_spec(HID)] * 6 + [_full_spec(w.shape), _full_spec(b.shape)],
        out_specs=[_row_spec(HID), _row_spec(HID)],
        out_shape=[jax.ShapeDtypeStruct((N, HID), jnp.float32)] * 2,
    )(x, h, pox, poh, pix, pih, w, b)


def _cand_call(x, q, pox, poq, pix, piq, z, h, w, b):
    return pl.pallas_call(
        _cand_body,
        grid=(_GRID,),
        in_specs=[_row_spec(HID)] * 8 + [_full_spec(w.shape), _full_spec(b.shape)],
        out_specs=_row_spec(HID),
        out_shape=jax.ShapeDtypeStruct((N, HID), jnp.float32),
    )(x, q, pox, poq, pix, piq, z, h, w, b)


def _final_linear_body(h_ref, w_ref, b_ref, o_ref):
    o_ref[...] = h_ref[...] @ w_ref[...] + b_ref[...][None, :]


def _pack_gate_weight(w):
    s = w[0, 0] + w[1, 0]
    return jnp.concatenate(
        [s[:HID], s[HID:], w[0, 1][:HID], w[0, 1][HID:],
         w[1, 1][:HID], w[1, 1][HID:]], axis=0)


def kernel(x, edge_index, edge_weight, w1_z, b1_z, w1_r, b1_r, w1_h, b1_h,
           w2_z, b2_z, w2_r, b2_r, w2_h, b2_h, lin_w, lin_b):
    B, n, f, T_ = x.shape
    E = edge_weight.shape[0]
    quant = NTILE * CH * 8
    ep = ((E + quant - 1) // quant) * quant
    pad = ep - E

    src = edge_index[0].astype(jnp.int32)
    dst = edge_index[1].astype(jnp.int32)
    zpi = jnp.arange(pad, dtype=jnp.int32) % N
    src_p = jnp.concatenate([src, zpi])
    dst_p = jnp.concatenate([dst, zpi])
    ew_p = jnp.concatenate([edge_weight, jnp.zeros((pad,), jnp.float32)])
    gidx = jnp.stack([src_p, dst_p])
    sidx = jnp.stack([dst_p, src_p])

    nrm = _make_norm_kernel(ep)(gidx, ew_p)

    prop_k = _make_prop_kernel(ep)

    def prop(v):
        out = prop_k(v, gidx, sidx, nrm)
        return out[0, :N], out[1, :N]

    xs = jnp.moveaxis(x[0], -1, 0)

    wz1 = jnp.concatenate([_pack_gate_weight(w1_z), _pack_gate_weight(w1_r)], axis=1)
    bz1 = jnp.concatenate([b1_z, b1_r]).reshape(1, 2 * HID)
    wh1 = _pack_gate_weight(w1_h)
    bh1 = b1_h.reshape(1, HID)
    wz2 = jnp.concatenate([_pack_gate_weight(w2_z), _pack_gate_weight(w2_r)], axis=1)
    bz2 = jnp.concatenate([b2_z, b2_r]).reshape(1, 2 * HID)
    wh2 = _pack_gate_weight(w2_h)
    bh2 = b2_h.reshape(1, HID)

    h1 = jnp.zeros((N, HID), jnp.float32)
    h2 = jnp.zeros((N, HID), jnp.float32)
    for t in range(T_):
        xt = xs[t]
        pox, pix = prop(xt)
        poh, pih = prop(h1)
        z1, q1 = _gates_call(xt, h1, pox, poh, pix, pih, wz1, bz1)
        poq, piq = prop(q1)
        h1 = _cand_call(xt, q1, pox, poq, pix, piq, z1, h1, wh1, bh1)

        pox2, pix2 = prop(h1)
        poh2, pih2 = prop(h2)
        z2, q2 = _gates_call(h1, h2, pox2, poh2, pix2, pih2, wz2, bz2)
        poq2, piq2 = prop(q2)
        h2 = _cand_call(h1, q2, pox2, poq2, pix2, piq2, z2, h2, wh2, bh2)

    out = pl.pallas_call(
        _final_linear_body,
        out_shape=jax.ShapeDtypeStruct((N, lin_w.shape[1]), jnp.float32),
    )(h2, lin_w, lin_b)
    return out.reshape(B, n, -1)
